# baseline (speedup 1.0000x reference)
import functools

import jax
import jax.numpy as jnp
from jax import lax
from jax.experimental import pallas as pl
from jax.experimental.pallas import tpu as pltpu
from jax.experimental.pallas import tpu_sc as plsc

_NUM_EXPERT = 64
_TOP_K = 2
_N = 16384
_IDX_DTYPE = jax.dtypes.canonicalize_dtype(jnp.int64)

_NW = 32
_CHUNK = 2048


def _sc_body(idx_hbm, gs_hbm, gsa_hbm, zi, hb, zb, ob, sem):
    c = lax.axis_index("c")
    s = lax.axis_index("s")
    wid = s * 2 + c

    zero_i = jnp.zeros((16,), jnp.int32)
    half_f = jnp.full((16,), 1.0 / _TOP_K, jnp.float32)
    zero_f = jnp.zeros((16,), jnp.float32)
    one_f = jnp.ones((16,), jnp.float32)
    for i in range(_CHUNK // 16):
        sl = pl.ds(i * 16, 16)
        if i < 1024 // 16:
            zi[sl] = zero_i
            hb[sl] = half_f
        zb[sl] = zero_f
        ob[sl] = one_f

    copies = []
    copies.append(pltpu.make_async_copy(
        zi, idx_hbm.at[pl.ds(wid * 1024, 1024)], sem))
    copies.append(pltpu.make_async_copy(
        hb, gs_hbm.at[pl.ds(wid * 1024, 1024)], sem))
    for j in range(16):
        base = (wid * 16 + j) * _CHUNK
        dst = gsa_hbm.at[pl.ds(base, _CHUNK)]
        is_ones = jnp.logical_and(wid == 0, j < _N // _CHUNK)
        ones_copy = pltpu.make_async_copy(ob, dst, sem)
        zeros_copy = pltpu.make_async_copy(zb, dst, sem)
        @pl.when(is_ones)
        def _start_ones(ones_copy=ones_copy):
            ones_copy.start()

        @pl.when(jnp.logical_not(is_ones))
        def _start_zeros(zeros_copy=zeros_copy):
            zeros_copy.start()
        copies.append(zeros_copy)
    copies[0].start()
    copies[1].start()
    for cp in copies:
        cp.wait()


@functools.partial(
    pl.kernel,
    mesh=plsc.VectorSubcoreMesh(core_axis_name="c", subcore_axis_name="s"),
    out_type=(
        jax.ShapeDtypeStruct((_N * _TOP_K,), _IDX_DTYPE),
        jax.ShapeDtypeStruct((_N * _TOP_K,), jnp.float32),
        jax.ShapeDtypeStruct((_NUM_EXPERT * _N,), jnp.float32),
    ),
    scratch_types=[
        pltpu.VMEM((1024,), jnp.int32),
        pltpu.VMEM((1024,), jnp.float32),
        pltpu.VMEM((_CHUNK,), jnp.float32),
        pltpu.VMEM((_CHUNK,), jnp.float32),
        pltpu.SemaphoreType.DMA,
    ],
)
def _sc_fill(idx_hbm, gs_hbm, gsa_hbm, zi, hb, zb, ob, sem):
    _sc_body(idx_hbm, gs_hbm, gsa_hbm, zi, hb, zb, ob, sem)


def kernel(inp):
    n = inp.shape[0]
    idx, gs_f, gsa_f = _sc_fill()
    gs = jnp.transpose(gs_f.reshape(_TOP_K, n), (1, 0)).reshape(n, 1, _TOP_K)
    gsa = jnp.transpose(gsa_f.reshape(_NUM_EXPERT, n), (1, 0))
    return idx, gs, gsa


# speedup vs baseline: 1.4666x; 1.4666x over previous
import functools

import jax
import jax.numpy as jnp
from jax import lax
from jax.experimental import pallas as pl
from jax.experimental.pallas import tpu as pltpu
from jax.experimental.pallas import tpu_sc as plsc

_NUM_EXPERT = 64
_TOP_K = 2
_N = 16384
_IDX_DTYPE = jax.dtypes.canonicalize_dtype(jnp.int64)


@functools.partial(
    pl.kernel,
    mesh=plsc.VectorSubcoreMesh(core_axis_name="c", subcore_axis_name="s"),
    out_type=jax.ShapeDtypeStruct((_N * _TOP_K,), _IDX_DTYPE),
    scratch_types=[
        pltpu.VMEM((1024,), jnp.int32),
        pltpu.SemaphoreType.DMA,
    ],
)
def _sc_idx(idx_hbm, zi, sem):
    c = lax.axis_index("c")
    s = lax.axis_index("s")
    wid = s * 2 + c
    zero_i = jnp.zeros((16,), jnp.int32)
    for i in range(64):
        zi[pl.ds(i * 16, 16)] = zero_i
    pltpu.make_async_copy(zi, idx_hbm.at[pl.ds(wid * 1024, 1024)], sem).start()
    pltpu.make_async_copy(zi, idx_hbm.at[pl.ds(wid * 1024, 1024)], sem).wait()


def _tc_body(gs_ref, gsa_ref):
    gs_ref[...] = jnp.full(gs_ref.shape, 1.0 / _TOP_K, jnp.float32)
    row = jax.lax.broadcasted_iota(jnp.int32, gsa_ref.shape, 0)
    gsa_ref[...] = (row == 0).astype(jnp.float32)


def kernel(inp):
    n = inp.shape[0]
    idx = _sc_idx()
    gs_t, gsa_t = pl.pallas_call(
        _tc_body,
        out_shape=(
            jax.ShapeDtypeStruct((_TOP_K, n), jnp.float32),
            jax.ShapeDtypeStruct((_NUM_EXPERT, n), jnp.float32),
        ),
    )()
    gs = jnp.transpose(gs_t, (1, 0)).reshape(n, 1, _TOP_K)
    gsa = jnp.transpose(gsa_t, (1, 0))
    return idx, gs, gsa


# speedup vs baseline: 11.6638x; 7.9529x over previous
import jax
import jax.numpy as jnp
from jax.experimental import pallas as pl
from jax.experimental.pallas import tpu as pltpu

_NUM_EXPERT = 64
_TOP_K = 2
_N = 16384
_IDX_DTYPE = jax.dtypes.canonicalize_dtype(jnp.int64)
_BLK = 8


def _fill_body(idx_hbm, gs_hbm, gsa_hbm, zi, hb, b0, zb, sem):
    zi[...] = jnp.zeros(zi.shape, _IDX_DTYPE)
    hb[...] = jnp.full(hb.shape, 1.0 / _TOP_K, jnp.float32)
    idx_cp = pltpu.make_async_copy(zi, idx_hbm, sem)
    gs_cp = pltpu.make_async_copy(hb, gs_hbm, sem)
    idx_cp.start()
    gs_cp.start()

    row = jax.lax.broadcasted_iota(jnp.int32, b0.shape, 0)
    b0[...] = (row == 0).astype(jnp.float32)
    zb[...] = jnp.zeros(zb.shape, jnp.float32)
    copies = [pltpu.make_async_copy(b0, gsa_hbm.at[pl.ds(0, _BLK)], sem)]
    for k in range(1, _NUM_EXPERT // _BLK):
        copies.append(
            pltpu.make_async_copy(zb, gsa_hbm.at[pl.ds(k * _BLK, _BLK)], sem))
    for cp in copies:
        cp.start()
    idx_cp.wait()
    gs_cp.wait()
    for cp in copies:
        cp.wait()


def kernel(inp):
    n = inp.shape[0]
    idx, gs_t, gsa_t = pl.pallas_call(
        _fill_body,
        out_shape=(
            jax.ShapeDtypeStruct((n * _TOP_K,), _IDX_DTYPE),
            jax.ShapeDtypeStruct((_TOP_K, n), jnp.float32),
            jax.ShapeDtypeStruct((_NUM_EXPERT, n), jnp.float32),
        ),
        out_specs=(
            pl.BlockSpec(memory_space=pltpu.ANY),
            pl.BlockSpec(memory_space=pltpu.ANY),
            pl.BlockSpec(memory_space=pltpu.ANY),
        ),
        scratch_shapes=[
            pltpu.VMEM((n * _TOP_K,), _IDX_DTYPE),
            pltpu.VMEM((_TOP_K, n), jnp.float32),
            pltpu.VMEM((_BLK, n), jnp.float32),
            pltpu.VMEM((_BLK, n), jnp.float32),
            pltpu.SemaphoreType.DMA,
        ],
    )()
    gs = jnp.transpose(gs_t, (1, 0)).reshape(n, 1, _TOP_K)
    gsa = jnp.transpose(gsa_t, (1, 0))
    return idx, gs, gsa


# speedup vs baseline: 11.7332x; 1.0059x over previous
import jax
import jax.numpy as jnp
from jax.experimental import pallas as pl
from jax.experimental.pallas import tpu as pltpu

_NUM_EXPERT = 64
_TOP_K = 2
_IDX_DTYPE = jax.dtypes.canonicalize_dtype(jnp.int64)
_BLK = 8


def _fill_body(idx_hbm, gs_hbm, gsa_hbm, zi, hb, b0, zb, sem):
    zi[...] = jnp.zeros(zi.shape, _IDX_DTYPE)
    hb[...] = jnp.full(hb.shape, 1.0 / _TOP_K, jnp.float32)
    idx_cp = pltpu.make_async_copy(zi, idx_hbm, sem)
    gs_cp = pltpu.make_async_copy(hb, gs_hbm, sem)
    idx_cp.start()
    gs_cp.start()

    row = jax.lax.broadcasted_iota(jnp.int32, b0.shape, 0)
    b0[...] = (row == 0).astype(jnp.float32)
    zb[...] = jnp.zeros(zb.shape, jnp.float32)
    copies = [pltpu.make_async_copy(b0, gsa_hbm.at[pl.ds(0, _BLK)], sem)]
    for k in range(1, _NUM_EXPERT // _BLK):
        copies.append(
            pltpu.make_async_copy(zb, gsa_hbm.at[pl.ds(k * _BLK, _BLK)], sem))
    for cp in copies:
        cp.start()
    idx_cp.wait()
    gs_cp.wait()
    for cp in copies:
        cp.wait()


def kernel(inp):
    n = inp.shape[0]
    idx, gs_t, gsa_t = pl.pallas_call(
        _fill_body,
        out_shape=(
            jax.ShapeDtypeStruct((n * _TOP_K,), _IDX_DTYPE),
            jax.ShapeDtypeStruct((_TOP_K, n), jnp.float32),
            jax.ShapeDtypeStruct((_NUM_EXPERT, n), jnp.float32),
        ),
        out_specs=(
            pl.BlockSpec(memory_space=pl.ANY),
            pl.BlockSpec(memory_space=pl.ANY),
            pl.BlockSpec(memory_space=pl.ANY),
        ),
        scratch_shapes=[
            pltpu.VMEM((n * _TOP_K,), _IDX_DTYPE),
            pltpu.VMEM((_TOP_K, n), jnp.float32),
            pltpu.VMEM((_BLK, n), jnp.float32),
            pltpu.VMEM((_BLK, n), jnp.float32),
            pltpu.SemaphoreType.DMA,
        ],
    )()
    gs = jnp.transpose(gs_t, (1, 0)).reshape(n, 1, _TOP_K)
    gsa = jnp.transpose(gsa_t, (1, 0))
    return idx, gs, gsa
